# Initial kernel scaffold; baseline (speedup 1.0000x reference)
import jax
import jax.numpy as jnp
from jax.experimental import pallas as pl


def kernel(state, edge_index, W_gcn, b_gcn, W1, b1, W2, b2, W3, b3, deterministic):
    raise NotImplementedError("write your pallas kernel here")


# speedup vs baseline: 3.6237x; 3.6237x over previous
import functools

import jax
import jax.numpy as jnp
from jax import lax
from jax.experimental import pallas as pl
from jax.experimental.pallas import tpu as pltpu
from jax.experimental.pallas import tpu_sc as plsc

N = 10000
E = 160000
C = 256
ACT = 8

NP = 10240
HALF = NP // 2
ROWS_PER_W = HALF // 16
EPW = E // 16
BATCH = 128
NB = (EPW + BATCH - 1) // BATCH
TRASH = HALF
DEG_ROWS = HALF + 8

NW = 32
TROWS = NP // NW
SB = 64
CE = 2000
NCHUNK = E // CE
NB_SEG = (CE + SB - 1) // SB
CAP_SEG = NB_SEG * SB
STRASH = TROWS
SACC_ROWS = TROWS + 8

def _deg_body(dst_hbm, deg_hbm, dst_w, idx2d, ones_v, zeros_v, deg_acc):
    c = lax.axis_index("c")
    s = lax.axis_index("s")
    lo = c * HALF

    for i in range(ROWS_PER_W // 16):
        zeros_v[pl.ds(i * 16, 16)] = jnp.zeros((16,), jnp.float32)
    pltpu.sync_copy(zeros_v, deg_acc.at[pl.ds(s * ROWS_PER_W, ROWS_PER_W)])
    for i in range(BATCH // 16):
        ones_v[pl.ds(i * 16, 16)] = jnp.ones((16,), jnp.float32)

    pltpu.sync_copy(dst_hbm.at[pl.ds(s * EPW, EPW)], dst_w)

    def scan_row(jr, carry):
        for k in range(8):
            dv = dst_w[pl.ds(jr * BATCH + k * 16, 16)]
            local = dv - lo
            ok = (local >= 0) & (local < HALF)
            idx2d[jr, pl.ds(k * 16, 16)] = jnp.where(ok, local, TRASH)
        return carry

    lax.fori_loop(0, NB - 1, scan_row, 0)
    dv = dst_w[pl.ds((NB - 1) * BATCH, 16)]
    local = dv - lo
    ok = (local >= 0) & (local < HALF)
    idx2d[NB - 1, pl.ds(0, 16)] = jnp.where(ok, local, TRASH)
    trash_v = jnp.full((16,), TRASH, jnp.int32)
    for k in range(1, 8):
        idx2d[NB - 1, pl.ds(k * 16, 16)] = trash_v

    plsc.subcore_barrier()
    def add_batch(j, carry):
        pltpu.sync_copy(ones_v, deg_acc.at[idx2d.at[j]], add=True)
        return carry

    lax.fori_loop(0, NB, add_batch, 0)
    plsc.subcore_barrier()
    pltpu.sync_copy(deg_acc.at[pl.ds(s * ROWS_PER_W, ROWS_PER_W)], zeros_v)
    pltpu.sync_copy(zeros_v, deg_hbm.at[pl.ds(lo + s * ROWS_PER_W, ROWS_PER_W)])


def _g_body(state_ref, deg_ref, w_ref, g_ref):
    dinv = lax.rsqrt(jnp.maximum(deg_ref[...] + 1.0, 1.0))
    h = jnp.dot(state_ref[...], w_ref[...], preferred_element_type=jnp.float32)
    g_ref[...] = h * dinv


RB = 1024


def _g_call(state_p, deg_col, w_gcn):
    return pl.pallas_call(
        _g_body,
        grid=(NP // RB,),
        in_specs=[
            pl.BlockSpec((RB, C), lambda i: (i, 0)),
            pl.BlockSpec((RB, 1), lambda i: (i, 0)),
            pl.BlockSpec((C, C), lambda i: (0, 0)),
        ],
        out_specs=pl.BlockSpec((RB, C), lambda i: (i, 0)),
        out_shape=jax.ShapeDtypeStruct((NP, C), jnp.float32),
    )(state_p, deg_col, w_gcn)


def _seg_body(src_hbm, dst_hbm, g_hbm, pad_src_hbm, pad_dst_hbm, out_hbm,
              src_w, dst_w, ks, kd, gbuf, acc, sem):
    c = lax.axis_index("c")
    s = lax.axis_index("s")
    wid = c * 16 + s
    lo = wid * TROWS

    pltpu.sync_copy(g_hbm.at[pl.ds(lo, TROWS)], acc.at[pl.ds(0, TROWS)])

    lane = lax.iota(jnp.int32, 16)

    def chunk_body(ci, carry):
        ebase = ci * CE
        pltpu.sync_copy(src_hbm.at[pl.ds(ebase, CE)], src_w)
        pltpu.sync_copy(dst_hbm.at[pl.ds(ebase, CE)], dst_w)
        pltpu.sync_copy(pad_src_hbm, ks)
        pltpu.sync_copy(pad_dst_hbm, kd)

        def compact(i, cnt):
            sv = src_w[pl.ds(i * 16, 16)]
            dv = dst_w[pl.ds(i * 16, 16)]
            local = dv - lo
            ok = (local >= 0) & (local < TROWS)
            ones = jnp.where(ok, 1, 0)
            pos = plsc.cumsum(ones)
            tgt = cnt + pos - 1
            row = lax.shift_right_logical(tgt, 6)
            col = tgt & (SB - 1)
            plsc.store_scatter(ks, [row, col], sv, mask=ok)
            plsc.store_scatter(kd, [row, col], local, mask=ok)
            return cnt + jnp.sum(ones)

        cnt = lax.fori_loop(0, CE // 16, compact, jnp.int32(0))
        nb = (cnt + SB - 1) // SB

        def flush(j, carry2):
            pltpu.async_copy(g_hbm.at[ks.at[j]], gbuf, sem).wait()

            def edge_add(e, carry3):
                rowv = plsc.load_gather(
                    kd, [jnp.broadcast_to(j, (16,)),
                         jnp.broadcast_to(e, (16,))])
                for k in range(C // 16):
                    v = gbuf[e, pl.ds(k * 16, 16)]
                    plsc.addupdate_scatter(acc, [rowv, lane + (k * 16)], v)
                return carry3

            lax.fori_loop(0, SB, edge_add, 0)
            return carry2

        lax.fori_loop(0, nb, flush, 0)
        return carry

    lax.fori_loop(0, NCHUNK, chunk_body, 0)

    pltpu.sync_copy(acc.at[pl.ds(0, TROWS)], out_hbm.at[pl.ds(lo, TROWS)])


def _mlp_body(s_ref, deg_ref, state_ref, bg_ref, w1_ref, b1_ref, w2_ref,
              b2_ref, w3_ref, b3_ref, act_ref, reg_ref):
    i = pl.program_id(0)
    dinv = lax.rsqrt(jnp.maximum(deg_ref[...] + 1.0, 1.0))
    x = jnp.maximum(s_ref[...] * dinv + bg_ref[...], 0.0) + state_ref[...]
    y = jnp.dot(x, w1_ref[...], preferred_element_type=jnp.float32) + b1_ref[...]
    y = jnp.where(y >= 0, y, 0.01 * y)
    y = jnp.dot(y, w2_ref[...], preferred_element_type=jnp.float32) + b2_ref[...]
    y = jnp.where(y >= 0, y, 0.01 * y)
    z = jnp.dot(y, w3_ref[...], preferred_element_type=jnp.float32) + b3_ref[...]
    sp = jnp.maximum(z, 0.0) + jnp.log1p(jnp.exp(-jnp.abs(z)))

    qr = lax.broadcasted_iota(jnp.int32, (RB, RB // ACT), 0) // ACT
    qc = lax.broadcasted_iota(jnp.int32, (RB, RB // ACT), 1)
    q = (qr == qc).astype(jnp.float32)
    gsum = jnp.dot(q, lax.dot_general(q, sp, (((0,), (0,)), ((), ()))),
                   preferred_element_type=jnp.float32)
    act_ref[...] = sp / (gsum + 1e-20)

    rows = i * RB + lax.broadcasted_iota(jnp.int32, (RB, 1), 0)
    part = jnp.sum(jnp.where(rows < N, jnp.abs(sp), 0.0), keepdims=True)

    @pl.when(i == 0)
    def _():
        reg_ref[...] = jnp.zeros((1, 1), jnp.float32)

    reg_ref[...] += part.reshape(1, 1)


def _mlp_call(s_mat, deg_col, state_p, bg, w1, b1, w2, b2, w3, b3):
    full = lambda r, c_: pl.BlockSpec((r, c_), lambda i: (0, 0))
    return pl.pallas_call(
        _mlp_body,
        grid=(NP // RB,),
        in_specs=[
            pl.BlockSpec((RB, C), lambda i: (i, 0)),
            pl.BlockSpec((RB, 1), lambda i: (i, 0)),
            pl.BlockSpec((RB, C), lambda i: (i, 0)),
            full(1, C), full(C, C), full(1, C), full(C, C), full(1, C),
            full(C, 1), full(1, 1),
        ],
        out_specs=[
            pl.BlockSpec((RB, 1), lambda i: (i, 0)),
            pl.BlockSpec((1, 1), lambda i: (0, 0)),
        ],
        out_shape=[
            jax.ShapeDtypeStruct((NP, 1), jnp.float32),
            jax.ShapeDtypeStruct((1, 1), jnp.float32),
        ],
    )(s_mat, deg_col, state_p, bg, w1, b1, w2, b2, w3, b3)


@functools.lru_cache(maxsize=1)
def _sc_kernels():
    mesh = plsc.VectorSubcoreMesh(core_axis_name="c", subcore_axis_name="s")
    params = pltpu.CompilerParams(needs_layout_passes=False)
    deg_kernel = pl.kernel(
        _deg_body,
        out_type=jax.ShapeDtypeStruct((NP,), jnp.float32),
        mesh=mesh,
        compiler_params=params,
        scratch_types=[
            pltpu.VMEM((EPW,), jnp.int32),
            pltpu.VMEM((NB, BATCH), jnp.int32),
            pltpu.VMEM((BATCH,), jnp.float32),
            pltpu.VMEM((ROWS_PER_W,), jnp.float32),
            pltpu.VMEM_SHARED((DEG_ROWS,), jnp.float32),
        ],
    )
    seg_kernel = pl.kernel(
        _seg_body,
        out_type=jax.ShapeDtypeStruct((NP, C), jnp.float32),
        mesh=mesh,
        compiler_params=params,
        scratch_types=[
            pltpu.VMEM((CE,), jnp.int32),
            pltpu.VMEM((CE,), jnp.int32),
            pltpu.VMEM((NB_SEG, SB), jnp.int32),
            pltpu.VMEM((NB_SEG, SB), jnp.int32),
            pltpu.VMEM((SB, C), jnp.float32),
            pltpu.VMEM((SACC_ROWS, C), jnp.float32),
            pltpu.SemaphoreType.DMA,
        ],
    )
    return deg_kernel, seg_kernel


def kernel(state, edge_index, W_gcn, b_gcn, W1, b1, W2, b2, W3, b3,
           deterministic=True):
    deg_kernel, seg_kernel = _sc_kernels()
    src = edge_index[0]
    dst = edge_index[1]
    state_p = jnp.pad(state, ((0, NP - N), (0, 0)))

    deg = deg_kernel(dst)
    deg_col = deg.reshape(NP, 1)

    g = _g_call(state_p, deg_col, W_gcn)

    ar = jnp.arange(CAP_SEG, dtype=jnp.int32)
    pad_src = ((ar * 5) % N).reshape(NB_SEG, SB)
    pad_dst = jnp.full((NB_SEG, SB), STRASH, jnp.int32)
    s_mat = seg_kernel(src, dst, g, pad_src, pad_dst)

    act_col, reg = _mlp_call(
        s_mat, deg_col, state_p,
        b_gcn.reshape(1, C), W1, b1.reshape(1, C), W2, b2.reshape(1, C),
        W3, b3.reshape(1, 1),
    )
    action = act_col[:N, 0].reshape(N // ACT, ACT)
    regularize = (reg[0, 0] / N).reshape(())
    return (action, regularize)


# speedup vs baseline: 4.9367x; 1.3623x over previous
import functools

import jax
import jax.numpy as jnp
from jax import lax
from jax.experimental import pallas as pl
from jax.experimental.pallas import tpu as pltpu
from jax.experimental.pallas import tpu_sc as plsc

N = 10000
E = 160000
C = 256
ACT = 8

NP = 10240
HALF = NP // 2
ROWS_PER_W = HALF // 16
EPW = E // 16
BATCH = 128
NB = (EPW + BATCH - 1) // BATCH
TRASH = HALF
DEG_ROWS = HALF + 8

NW = 32
TROWS = NP // NW
SB = 64
CE = 3200
NCHUNK = E // CE
NB_SEG = CE // SB + 1
STRASH = TROWS
SACC_ROWS = TROWS + 8

def _deg_body(dst_hbm, deg_hbm, dst_w, idx2d, ones_v, zeros_v, deg_acc):
    c = lax.axis_index("c")
    s = lax.axis_index("s")
    lo = c * HALF

    for i in range(ROWS_PER_W // 16):
        zeros_v[pl.ds(i * 16, 16)] = jnp.zeros((16,), jnp.float32)
    pltpu.sync_copy(zeros_v, deg_acc.at[pl.ds(s * ROWS_PER_W, ROWS_PER_W)])
    for i in range(BATCH // 16):
        ones_v[pl.ds(i * 16, 16)] = jnp.ones((16,), jnp.float32)

    pltpu.sync_copy(dst_hbm.at[pl.ds(s * EPW, EPW)], dst_w)

    def scan_row(jr, carry):
        for k in range(8):
            dv = dst_w[pl.ds(jr * BATCH + k * 16, 16)]
            local = dv - lo
            ok = (local >= 0) & (local < HALF)
            idx2d[jr, pl.ds(k * 16, 16)] = jnp.where(ok, local, TRASH)
        return carry

    lax.fori_loop(0, NB - 1, scan_row, 0)
    dv = dst_w[pl.ds((NB - 1) * BATCH, 16)]
    local = dv - lo
    ok = (local >= 0) & (local < HALF)
    idx2d[NB - 1, pl.ds(0, 16)] = jnp.where(ok, local, TRASH)
    trash_v = jnp.full((16,), TRASH, jnp.int32)
    for k in range(1, 8):
        idx2d[NB - 1, pl.ds(k * 16, 16)] = trash_v

    plsc.subcore_barrier()
    def add_batch(j, carry):
        pltpu.sync_copy(ones_v, deg_acc.at[idx2d.at[j]], add=True)
        return carry

    lax.fori_loop(0, NB, add_batch, 0)
    plsc.subcore_barrier()
    pltpu.sync_copy(deg_acc.at[pl.ds(s * ROWS_PER_W, ROWS_PER_W)], zeros_v)
    pltpu.sync_copy(zeros_v, deg_hbm.at[pl.ds(lo + s * ROWS_PER_W, ROWS_PER_W)])


def _g_body(state_ref, deg_ref, w_ref, g_ref):
    dinv = lax.rsqrt(jnp.maximum(deg_ref[...] + 1.0, 1.0))
    h = jnp.dot(state_ref[...], w_ref[...], preferred_element_type=jnp.float32)
    g_ref[...] = h * dinv


RB = 1024


def _g_call(state_p, deg_col, w_gcn):
    return pl.pallas_call(
        _g_body,
        grid=(NP // RB,),
        in_specs=[
            pl.BlockSpec((RB, C), lambda i: (i, 0)),
            pl.BlockSpec((RB, 1), lambda i: (i, 0)),
            pl.BlockSpec((C, C), lambda i: (0, 0)),
        ],
        out_specs=pl.BlockSpec((RB, C), lambda i: (i, 0)),
        out_shape=jax.ShapeDtypeStruct((NP, C), jnp.float32),
    )(state_p, deg_col, w_gcn)


def _seg_body(src_hbm, dst_hbm, g_hbm, out_hbm,
              src_w, dst_w, ks, kd, gbuf, acc, sem, sem_s, sem_d):
    c = lax.axis_index("c")
    s = lax.axis_index("s")
    wid = c * 16 + s
    lo = wid * TROWS

    pltpu.sync_copy(g_hbm.at[pl.ds(lo, TROWS)], acc.at[pl.ds(0, TROWS)])

    lane = lax.iota(jnp.int32, 16)

    pltpu.async_copy(src_hbm.at[pl.ds(0, CE)], src_w.at[0], sem_s)
    pltpu.async_copy(dst_hbm.at[pl.ds(0, CE)], dst_w.at[0], sem_d)

    def chunk_body(ci, carry):
        b = ci & 1
        ebase = ci * CE
        pltpu.make_async_copy(src_hbm.at[pl.ds(ebase, CE)],
                              src_w.at[b], sem_s).wait()
        pltpu.make_async_copy(dst_hbm.at[pl.ds(ebase, CE)],
                              dst_w.at[b], sem_d).wait()

        @pl.when(ci + 1 < NCHUNK)
        def _():
            nbase = ebase + CE
            pltpu.async_copy(src_hbm.at[pl.ds(nbase, CE)],
                             src_w.at[1 - b], sem_s)
            pltpu.async_copy(dst_hbm.at[pl.ds(nbase, CE)],
                             dst_w.at[1 - b], sem_d)

        def compact(i, cntv):
            sv = src_w[b, pl.ds(i * 16, 16)]
            dv = dst_w[b, pl.ds(i * 16, 16)]
            local = dv - lo
            ok = (local >= 0) & (local < TROWS)
            pos = plsc.cumsum(jnp.where(ok, 1, 0))
            tgt = cntv + pos - 1
            row = lax.shift_right_logical(tgt, 6)
            col = tgt & (SB - 1)
            plsc.store_scatter(ks, [row, col], sv, mask=ok)
            plsc.store_scatter(kd, [row, col], local, mask=ok)
            return cntv + plsc.all_reduce_population_count(ok)

        cntv = lax.fori_loop(0, CE // 16, compact,
                             jnp.zeros((16,), jnp.int32))
        for t in range(4):
            tgt = cntv + (t * 16) + lane
            row = lax.shift_right_logical(tgt, 6)
            col = tgt & (SB - 1)
            plsc.store_scatter(ks, [row, col], lo + lane)
            plsc.store_scatter(kd, [row, col],
                               jnp.broadcast_to(STRASH, (16,)))
        cnt = jnp.max(cntv)
        nb = (cnt + SB - 1) // SB

        def flush(j, carry2):
            pltpu.async_copy(g_hbm.at[ks.at[j]], gbuf, sem).wait()

            def edge_add(e, carry3):
                rowv = plsc.load_gather(
                    kd, [jnp.broadcast_to(j, (16,)),
                         jnp.broadcast_to(e, (16,))])
                for k in range(C // 16):
                    v = gbuf[e, pl.ds(k * 16, 16)]
                    plsc.addupdate_scatter(acc, [rowv, lane + (k * 16)], v)
                return carry3

            lax.fori_loop(0, SB, edge_add, 0)
            return carry2

        lax.fori_loop(0, nb, flush, 0)
        return carry

    lax.fori_loop(0, NCHUNK, chunk_body, 0)

    pltpu.sync_copy(acc.at[pl.ds(0, TROWS)], out_hbm.at[pl.ds(lo, TROWS)])


def _mlp_body(s_ref, deg_ref, state_ref, bg_ref, w1_ref, b1_ref, w2_ref,
              b2_ref, w3_ref, b3_ref, act_ref, reg_ref):
    i = pl.program_id(0)
    dinv = lax.rsqrt(jnp.maximum(deg_ref[...] + 1.0, 1.0))
    x = jnp.maximum(s_ref[...] * dinv + bg_ref[...], 0.0) + state_ref[...]
    y = jnp.dot(x, w1_ref[...], preferred_element_type=jnp.float32) + b1_ref[...]
    y = jnp.where(y >= 0, y, 0.01 * y)
    y = jnp.dot(y, w2_ref[...], preferred_element_type=jnp.float32) + b2_ref[...]
    y = jnp.where(y >= 0, y, 0.01 * y)
    z = jnp.dot(y, w3_ref[...], preferred_element_type=jnp.float32) + b3_ref[...]
    sp = jnp.maximum(z, 0.0) + jnp.log1p(jnp.exp(-jnp.abs(z)))

    qr = lax.broadcasted_iota(jnp.int32, (RB, RB // ACT), 0) // ACT
    qc = lax.broadcasted_iota(jnp.int32, (RB, RB // ACT), 1)
    q = (qr == qc).astype(jnp.float32)
    gsum = jnp.dot(q, lax.dot_general(q, sp, (((0,), (0,)), ((), ()))),
                   preferred_element_type=jnp.float32)
    act_ref[...] = sp / (gsum + 1e-20)

    rows = i * RB + lax.broadcasted_iota(jnp.int32, (RB, 1), 0)
    part = jnp.sum(jnp.where(rows < N, jnp.abs(sp), 0.0), keepdims=True)

    @pl.when(i == 0)
    def _():
        reg_ref[...] = jnp.zeros((1, 1), jnp.float32)

    reg_ref[...] += part.reshape(1, 1)


def _mlp_call(s_mat, deg_col, state_p, bg, w1, b1, w2, b2, w3, b3):
    full = lambda r, c_: pl.BlockSpec((r, c_), lambda i: (0, 0))
    return pl.pallas_call(
        _mlp_body,
        grid=(NP // RB,),
        in_specs=[
            pl.BlockSpec((RB, C), lambda i: (i, 0)),
            pl.BlockSpec((RB, 1), lambda i: (i, 0)),
            pl.BlockSpec((RB, C), lambda i: (i, 0)),
            full(1, C), full(C, C), full(1, C), full(C, C), full(1, C),
            full(C, 1), full(1, 1),
        ],
        out_specs=[
            pl.BlockSpec((RB, 1), lambda i: (i, 0)),
            pl.BlockSpec((1, 1), lambda i: (0, 0)),
        ],
        out_shape=[
            jax.ShapeDtypeStruct((NP, 1), jnp.float32),
            jax.ShapeDtypeStruct((1, 1), jnp.float32),
        ],
    )(s_mat, deg_col, state_p, bg, w1, b1, w2, b2, w3, b3)


@functools.lru_cache(maxsize=1)
def _sc_kernels():
    mesh = plsc.VectorSubcoreMesh(core_axis_name="c", subcore_axis_name="s")
    params = pltpu.CompilerParams(needs_layout_passes=False)
    deg_kernel = pl.kernel(
        _deg_body,
        out_type=jax.ShapeDtypeStruct((NP,), jnp.float32),
        mesh=mesh,
        compiler_params=params,
        scratch_types=[
            pltpu.VMEM((EPW,), jnp.int32),
            pltpu.VMEM((NB, BATCH), jnp.int32),
            pltpu.VMEM((BATCH,), jnp.float32),
            pltpu.VMEM((ROWS_PER_W,), jnp.float32),
            pltpu.VMEM_SHARED((DEG_ROWS,), jnp.float32),
        ],
    )
    seg_kernel = pl.kernel(
        _seg_body,
        out_type=jax.ShapeDtypeStruct((NP, C), jnp.float32),
        mesh=mesh,
        compiler_params=params,
        scratch_types=[
            pltpu.VMEM((2, CE), jnp.int32),
            pltpu.VMEM((2, CE), jnp.int32),
            pltpu.VMEM((NB_SEG, SB), jnp.int32),
            pltpu.VMEM((NB_SEG, SB), jnp.int32),
            pltpu.VMEM((SB, C), jnp.float32),
            pltpu.VMEM((SACC_ROWS, C), jnp.float32),
            pltpu.SemaphoreType.DMA,
            pltpu.SemaphoreType.DMA,
            pltpu.SemaphoreType.DMA,
        ],
    )
    return deg_kernel, seg_kernel


def kernel(state, edge_index, W_gcn, b_gcn, W1, b1, W2, b2, W3, b3,
           deterministic=True):
    deg_kernel, seg_kernel = _sc_kernels()
    src = edge_index[0]
    dst = edge_index[1]
    state_p = jnp.pad(state, ((0, NP - N), (0, 0)))

    deg = deg_kernel(dst)
    deg_col = deg.reshape(NP, 1)

    g = _g_call(state_p, deg_col, W_gcn)

    s_mat = seg_kernel(src, dst, g)

    act_col, reg = _mlp_call(
        s_mat, deg_col, state_p,
        b_gcn.reshape(1, C), W1, b1.reshape(1, C), W2, b2.reshape(1, C),
        W3, b3.reshape(1, 1),
    )
    action = act_col[:N, 0].reshape(N // ACT, ACT)
    regularize = (reg[0, 0] / N).reshape(())
    return (action, regularize)
